# baseline (speedup 1.0000x reference)
import functools

import jax
import jax.numpy as jnp
from jax import lax
from jax.experimental import pallas as pl
from jax.experimental.pallas import tpu as pltpu


def _gram_kernel(feat_ref, out_ref, *, half, nsteps, scale, tk, k_total, ragged):
    i = pl.program_id(0)
    kk = pl.program_id(1)

    @pl.when(kk == 0)
    def _():
        out_ref[...] = jnp.zeros_like(out_ref)

    rhs = feat_ref[...]
    lhs = feat_ref[pl.ds(i * half, half), :]
    if ragged:
        valid = k_total - kk * tk
        col = lax.broadcasted_iota(jnp.int32, rhs.shape, 1)
        rhs = jnp.where(col < valid, rhs, jnp.zeros_like(rhs))
        lcol = lax.broadcasted_iota(jnp.int32, lhs.shape, 1)
        lhs = jnp.where(lcol < valid, lhs, jnp.zeros_like(lhs))
    rhs = rhs.astype(jnp.bfloat16)
    lhs = lhs.astype(jnp.bfloat16)

    out_ref[...] += lax.dot_general(
        lhs, rhs,
        dimension_numbers=(((1,), (1,)), ((), ())),
        preferred_element_type=jnp.float32,
    )

    @pl.when(kk == nsteps - 1)
    def _():
        out_ref[...] = out_ref[...] * scale


def _tiling(m, k, itemsize, panel_bytes):
    cap = max(128, panel_bytes // max(m * itemsize, 1))
    cap -= cap % 128
    if k <= cap:
        return k, 1, False
    for tk in range(cap, 127, -128):
        if k % tk == 0:
            return tk, k // tk, False
    return cap, pl.cdiv(k, cap), True


def kernel(x):
    b, c, h, w = x.shape
    m, k = b * c, h * w
    feats = x.reshape(m, k)
    scale = 1.0 / float(b * c * h * w)

    tk, steps, ragged = _tiling(m, k, feats.dtype.itemsize, 4 << 20)
    nh = 2 if (m % 16 == 0) else 1
    half = m // nh

    vmem = 2 * m * tk * feats.dtype.itemsize + half * m * 4
    limit = int(min(max(vmem + (16 << 20), 32 << 20), 64 << 20))

    return pl.pallas_call(
        functools.partial(_gram_kernel, half=half, nsteps=steps, scale=scale,
                          tk=tk, k_total=k, ragged=ragged),
        out_shape=jax.ShapeDtypeStruct((m, m), jnp.float32),
        grid=(nh, steps),
        in_specs=[pl.BlockSpec((m, tk), lambda i, kk: (0, kk))],
        out_specs=pl.BlockSpec((half, m), lambda i, kk: (i, 0)),
        compiler_params=pltpu.CompilerParams(
            dimension_semantics=("parallel", "arbitrary"),
            vmem_limit_bytes=limit,
        ),
    )(feats)


# speedup vs baseline: 3.8264x; 3.8264x over previous
import functools

import jax
import jax.numpy as jnp
from jax import lax
from jax.experimental import pallas as pl
from jax.experimental.pallas import tpu as pltpu


def _gram_kernel(feat_ref, out_ref, *, nsteps, scale):
    kk = pl.program_id(1)

    @pl.when(kk == 0)
    def _():
        out_ref[...] = jnp.zeros_like(out_ref)

    f = feat_ref[...].astype(jnp.bfloat16)
    f = f.reshape(f.shape[0], f.shape[1] * f.shape[2])
    out_ref[...] += lax.dot_general(
        f, f,
        dimension_numbers=(((1,), (1,)), ((), ())),
        preferred_element_type=jnp.float32,
    )

    @pl.when(kk == nsteps - 1)
    def _():
        out_ref[...] = out_ref[...] * scale


def kernel(x):
    b, c, h, w = x.shape
    m = b * c
    feats = x.reshape(m, h, w)
    scale = 1.0 / float(b * c * h * w)

    th = 16
    while h % th:
        th //= 2
    steps = h // th

    return pl.pallas_call(
        functools.partial(_gram_kernel, nsteps=steps, scale=scale),
        out_shape=jax.ShapeDtypeStruct((m, m), jnp.float32),
        grid=(1, steps),
        in_specs=[pl.BlockSpec((m, th, w), lambda i, kk: (0, kk, 0))],
        out_specs=pl.BlockSpec((m, m), lambda i, kk: (0, 0)),
        compiler_params=pltpu.CompilerParams(
            dimension_semantics=("parallel", "arbitrary"),
            vmem_limit_bytes=64 << 20,
        ),
    )(feats)
